# baseline (speedup 1.0000x reference)
import functools

import jax
import jax.numpy as jnp
from jax import lax
from jax.experimental import pallas as pl
from jax.experimental.pallas import tpu as pltpu
from jax.experimental.pallas import tpu_sc as plsc

NUM_CLASSES = 10
FEAT_DIM = 2
ALPHA = 0.5
BATCH = 16384

NUM_TILES = 16
CHUNK = BATCH // NUM_TILES
LANES = 16
STEPS = CHUNK // LANES
PART = 3 * LANES
CEN = NUM_CLASSES * FEAT_DIM


def _permute(vals, idx):
    return jnp.take_along_axis(
        vals, idx, axis=0, mode=lax.GatherScatterMode.PROMISE_IN_BOUNDS
    )


def _make_kernel():
    mesh = plsc.VectorSubcoreMesh(
        core_axis_name="c", subcore_axis_name="s", num_cores=1
    )

    @functools.partial(
        pl.kernel,
        mesh=mesh,
        compiler_params=pltpu.CompilerParams(needs_layout_passes=False),
        out_type=[
            jax.ShapeDtypeStruct((BATCH,), jnp.float32),
            jax.ShapeDtypeStruct((CEN,), jnp.float32),
        ],
        scratch_types=[
            pltpu.VMEM((CHUNK,), jnp.int32),
            pltpu.VMEM((CHUNK * FEAT_DIM,), jnp.float32),
            pltpu.VMEM((CHUNK,), jnp.float32),
            pltpu.VMEM((2 * LANES,), jnp.float32),
            pltpu.VMEM((2 * PART,), jnp.float32),
            pltpu.VMEM((NUM_TILES * PART,), jnp.float32),
            pltpu.VMEM((2 * LANES,), jnp.float32),
            pltpu.VMEM_SHARED((NUM_TILES * PART,), jnp.float32),
        ],
    )
    def k(feat_hbm, lab_hbm, cen_hbm, res_hbm, nc_hbm,
          lab_v, feat_v, res_v, cen_v, acc_v, all_v, nc_v, shared):
        wid = lax.axis_index("s")
        base = wid * CHUNK

        pltpu.sync_copy(lab_hbm.at[pl.ds(base, CHUNK)], lab_v)
        pltpu.sync_copy(feat_hbm.at[pl.ds(base, CHUNK)], feat_v.at[pl.ds(0, CHUNK)])
        pltpu.sync_copy(
            feat_hbm.at[pl.ds(BATCH + base, CHUNK)],
            feat_v.at[pl.ds(CHUNK, CHUNK)],
        )
        pltpu.sync_copy(cen_hbm, cen_v)

        iota = lax.iota(jnp.int32, LANES)
        cen0 = cen_v[pl.ds(0, LANES)]
        cen1 = cen_v[pl.ds(LANES, LANES)]
        ones_f = jnp.ones((LANES,), jnp.float32)
        zeros_f = jnp.zeros((LANES,), jnp.float32)

        for r in range(6):
            acc_v[pl.ds(r * LANES, LANES)] = zeros_f

        def step(j, _):
            for u in range(2):
                off = pl.multiple_of(j * (2 * LANES) + u * LANES, LANES)
                bank = u * PART
                lab = lab_v[pl.ds(off, LANES)]
                f0 = feat_v[pl.ds(off, LANES)]
                f1 = feat_v[pl.ds(CHUNK + off, LANES)]
                g0 = _permute(cen0, lab)
                g1 = _permute(cen1, lab)
                d0 = f0 - g0
                d1 = f1 - g1
                res_v[pl.ds(off, LANES)] = d0 * d0 + d1 * d1
                plsc.addupdate_scatter(acc_v, [lab + bank], ones_f)
                plsc.addupdate_scatter(acc_v, [lab + (bank + LANES)], f0)
                plsc.addupdate_scatter(acc_v, [lab + (bank + 2 * LANES)], f1)
            return 0

        lax.fori_loop(0, STEPS // 2, step, 0)

        for r in range(3):
            acc_v[pl.ds(r * LANES, LANES)] = (
                acc_v[pl.ds(r * LANES, LANES)]
                + acc_v[pl.ds(PART + r * LANES, LANES)]
            )

        pltpu.sync_copy(res_v, res_hbm.at[pl.ds(base, CHUNK)])

        pltpu.sync_copy(acc_v.at[pl.ds(0, PART)], shared.at[pl.ds(wid * PART, PART)])
        plsc.subcore_barrier()

        @pl.when(wid == 0)
        def _():
            pltpu.sync_copy(shared, all_v)
            cnt = zeros_f
            s0 = zeros_f
            s1 = zeros_f
            for t in range(NUM_TILES):
                cnt = cnt + all_v[pl.ds(t * PART, LANES)]
                s0 = s0 + all_v[pl.ds(t * PART + LANES, LANES)]
                s1 = s1 + all_v[pl.ds(t * PART + 2 * LANES, LANES)]
            valid = iota < NUM_CLASSES
            c0 = cen0
            c1 = cen1
            scale = ALPHA / (cnt + 1.0)
            n0 = c0 - (cnt * c0 - s0) * scale
            n1 = c1 - (cnt * c1 - s1) * scale
            plsc.store_scatter(nc_v, [iota], n0, mask=valid)
            plsc.store_scatter(nc_v, [iota + NUM_CLASSES], n1, mask=valid)
            pltpu.sync_copy(nc_v.at[pl.ds(0, CEN)], nc_hbm)

    return k


_sc_center_loss = _make_kernel()


@jax.jit
def kernel(features, labels, centers):
    cen_pad = jnp.zeros((FEAT_DIM, LANES), jnp.float32)
    cen_pad = cen_pad.at[:, :NUM_CLASSES].set(centers.T)
    res, nc = _sc_center_loss(
        features.T.reshape(-1), labels.reshape(-1), cen_pad.reshape(-1)
    )
    return res.reshape(-1, 1), nc.reshape(FEAT_DIM, NUM_CLASSES).T


# speedup vs baseline: 1.0615x; 1.0615x over previous
import functools

import jax
import jax.numpy as jnp
from jax import lax
from jax.experimental import pallas as pl
from jax.experimental.pallas import tpu as pltpu
from jax.experimental.pallas import tpu_sc as plsc

NUM_CLASSES = 10
FEAT_DIM = 2
ALPHA = 0.5
BATCH = 16384

NUM_TILES = 16
CHUNK = BATCH // NUM_TILES
LANES = 16
STEPS = CHUNK // LANES
PART = 3 * LANES
CEN = NUM_CLASSES * FEAT_DIM


def _permute(vals, idx):
    return jnp.take_along_axis(
        vals, idx, axis=0, mode=lax.GatherScatterMode.PROMISE_IN_BOUNDS
    )


def _make_kernel():
    mesh = plsc.VectorSubcoreMesh(
        core_axis_name="c", subcore_axis_name="s", num_cores=1
    )

    @functools.partial(
        pl.kernel,
        mesh=mesh,
        compiler_params=pltpu.CompilerParams(needs_layout_passes=False),
        out_type=[
            jax.ShapeDtypeStruct((BATCH,), jnp.float32),
            jax.ShapeDtypeStruct((CEN,), jnp.float32),
        ],
        scratch_types=[
            pltpu.VMEM((CHUNK,), jnp.int32),
            pltpu.VMEM((CHUNK * FEAT_DIM,), jnp.float32),
            pltpu.VMEM((CHUNK,), jnp.float32),
            pltpu.VMEM((2 * LANES,), jnp.float32),
            pltpu.VMEM((2 * PART,), jnp.float32),
            pltpu.VMEM((NUM_TILES * PART,), jnp.float32),
            pltpu.VMEM((2 * LANES,), jnp.float32),
            pltpu.VMEM_SHARED((NUM_TILES * PART,), jnp.float32),
            pltpu.SemaphoreType.DMA,
            pltpu.SemaphoreType.DMA,
        ],
    )
    def k(feat_hbm, lab_hbm, cen_hbm, res_hbm, nc_hbm,
          lab_v, feat_v, res_v, cen_v, acc_v, all_v, nc_v, shared,
          sem_in, sem_out):
        wid = lax.axis_index("s")
        base = wid * CHUNK

        h_lab = pltpu.async_copy(lab_hbm.at[pl.ds(base, CHUNK)], lab_v, sem_in)
        h_f0 = pltpu.async_copy(
            feat_hbm.at[pl.ds(base, CHUNK)], feat_v.at[pl.ds(0, CHUNK)], sem_in
        )
        h_f1 = pltpu.async_copy(
            feat_hbm.at[pl.ds(BATCH + base, CHUNK)],
            feat_v.at[pl.ds(CHUNK, CHUNK)],
            sem_in,
        )
        h_cen = pltpu.async_copy(cen_hbm, cen_v, sem_in)
        h_lab.wait()
        h_f0.wait()
        h_f1.wait()
        h_cen.wait()

        iota = lax.iota(jnp.int32, LANES)
        cen0 = cen_v[pl.ds(0, LANES)]
        cen1 = cen_v[pl.ds(LANES, LANES)]
        ones_f = jnp.ones((LANES,), jnp.float32)
        zeros_f = jnp.zeros((LANES,), jnp.float32)

        for r in range(6):
            acc_v[pl.ds(r * LANES, LANES)] = zeros_f

        def step(j, _):
            for u in range(2):
                off = pl.multiple_of(j * (2 * LANES) + u * LANES, LANES)
                bank = u * PART
                lab = lab_v[pl.ds(off, LANES)]
                f0 = feat_v[pl.ds(off, LANES)]
                f1 = feat_v[pl.ds(CHUNK + off, LANES)]
                g0 = _permute(cen0, lab)
                g1 = _permute(cen1, lab)
                d0 = f0 - g0
                d1 = f1 - g1
                res_v[pl.ds(off, LANES)] = d0 * d0 + d1 * d1
                plsc.addupdate_scatter(acc_v, [lab + bank], ones_f)
                plsc.addupdate_scatter(acc_v, [lab + (bank + LANES)], f0)
                plsc.addupdate_scatter(acc_v, [lab + (bank + 2 * LANES)], f1)
            return 0

        lax.fori_loop(0, STEPS // 2, step, 0)

        for r in range(3):
            acc_v[pl.ds(r * LANES, LANES)] = (
                acc_v[pl.ds(r * LANES, LANES)]
                + acc_v[pl.ds(PART + r * LANES, LANES)]
            )

        h_res = pltpu.async_copy(res_v, res_hbm.at[pl.ds(base, CHUNK)], sem_out)

        pltpu.sync_copy(acc_v.at[pl.ds(0, PART)], shared.at[pl.ds(wid * PART, PART)])
        plsc.subcore_barrier()

        @pl.when(wid == 0)
        def _():
            pltpu.sync_copy(shared, all_v)
            cnt = zeros_f
            s0 = zeros_f
            s1 = zeros_f
            for t in range(NUM_TILES):
                cnt = cnt + all_v[pl.ds(t * PART, LANES)]
                s0 = s0 + all_v[pl.ds(t * PART + LANES, LANES)]
                s1 = s1 + all_v[pl.ds(t * PART + 2 * LANES, LANES)]
            valid = iota < NUM_CLASSES
            c0 = cen0
            c1 = cen1
            scale = ALPHA / (cnt + 1.0)
            n0 = c0 - (cnt * c0 - s0) * scale
            n1 = c1 - (cnt * c1 - s1) * scale
            plsc.store_scatter(nc_v, [iota], n0, mask=valid)
            plsc.store_scatter(nc_v, [iota + NUM_CLASSES], n1, mask=valid)
            pltpu.sync_copy(nc_v.at[pl.ds(0, CEN)], nc_hbm)

        h_res.wait()

    return k


_sc_center_loss = _make_kernel()


@jax.jit
def kernel(features, labels, centers):
    cen_pad = jnp.zeros((FEAT_DIM, LANES), jnp.float32)
    cen_pad = cen_pad.at[:, :NUM_CLASSES].set(centers.T)
    res, nc = _sc_center_loss(
        features.T.reshape(-1), labels.reshape(-1), cen_pad.reshape(-1)
    )
    return res.reshape(-1, 1), nc.reshape(FEAT_DIM, NUM_CLASSES).T


# speedup vs baseline: 1.0796x; 1.0170x over previous
import functools

import jax
import jax.numpy as jnp
from jax import lax
from jax.experimental import pallas as pl
from jax.experimental.pallas import tpu as pltpu
from jax.experimental.pallas import tpu_sc as plsc

NUM_CLASSES = 10
FEAT_DIM = 2
ALPHA = 0.5
BATCH = 16384

NUM_TILES = 16
CHUNK = BATCH // NUM_TILES
LANES = 16
STEPS = CHUNK // LANES
PART = 3 * LANES
CEN = NUM_CLASSES * FEAT_DIM


def _permute(vals, idx):
    return jnp.take_along_axis(
        vals, idx, axis=0, mode=lax.GatherScatterMode.PROMISE_IN_BOUNDS
    )


def _make_kernel():
    mesh = plsc.VectorSubcoreMesh(
        core_axis_name="c", subcore_axis_name="s", num_cores=1
    )

    @functools.partial(
        pl.kernel,
        mesh=mesh,
        compiler_params=pltpu.CompilerParams(needs_layout_passes=False),
        out_type=[
            jax.ShapeDtypeStruct((BATCH,), jnp.float32),
            jax.ShapeDtypeStruct((CEN,), jnp.float32),
        ],
        scratch_types=[
            pltpu.VMEM((CHUNK,), jnp.int32),
            pltpu.VMEM((CHUNK * FEAT_DIM,), jnp.float32),
            pltpu.VMEM((CHUNK,), jnp.float32),
            pltpu.VMEM((2 * LANES,), jnp.float32),
            pltpu.VMEM((2 * PART,), jnp.float32),
            pltpu.VMEM((NUM_TILES * PART,), jnp.float32),
            pltpu.VMEM((2 * LANES,), jnp.float32),
            pltpu.VMEM_SHARED((NUM_TILES * PART,), jnp.float32),
            pltpu.SemaphoreType.DMA,
            pltpu.SemaphoreType.DMA,
        ],
    )
    def k(feat_hbm, lab_hbm, cen_hbm, res_hbm, nc_hbm,
          lab_v, feat_v, res_v, cen_v, acc_v, all_v, nc_v, shared,
          sem_in, sem_out):
        wid = lax.axis_index("s")
        base = wid * CHUNK

        h_lab = pltpu.async_copy(lab_hbm.at[pl.ds(base, CHUNK)], lab_v, sem_in)
        h_f0 = pltpu.async_copy(
            feat_hbm.at[pl.ds(base, CHUNK)], feat_v.at[pl.ds(0, CHUNK)], sem_in
        )
        h_f1 = pltpu.async_copy(
            feat_hbm.at[pl.ds(BATCH + base, CHUNK)],
            feat_v.at[pl.ds(CHUNK, CHUNK)],
            sem_in,
        )
        h_cen = pltpu.async_copy(cen_hbm, cen_v, sem_in)
        h_lab.wait()
        h_f0.wait()
        h_f1.wait()
        h_cen.wait()

        iota = lax.iota(jnp.int32, LANES)
        cen0 = cen_v[pl.ds(0, LANES)]
        cen1 = cen_v[pl.ds(LANES, LANES)]
        ones_f = jnp.ones((LANES,), jnp.float32)
        zeros_f = jnp.zeros((LANES,), jnp.float32)

        for r in range(6):
            acc_v[pl.ds(r * LANES, LANES)] = zeros_f

        def step(j, _):
            for u in range(2):
                off = pl.multiple_of(j * (2 * LANES) + u * LANES, LANES)
                bank = u * PART
                lab = lab_v[pl.ds(off, LANES)]
                f0 = feat_v[pl.ds(off, LANES)]
                f1 = feat_v[pl.ds(CHUNK + off, LANES)]
                g0 = _permute(cen0, lab)
                g1 = _permute(cen1, lab)
                d0 = f0 - g0
                d1 = f1 - g1
                res_v[pl.ds(off, LANES)] = d0 * d0 + d1 * d1
                plsc.addupdate_scatter(acc_v, [lab + bank], ones_f)
                plsc.addupdate_scatter(acc_v, [lab + (bank + LANES)], f0)
                plsc.addupdate_scatter(acc_v, [lab + (bank + 2 * LANES)], f1)
            return 0

        lax.fori_loop(0, STEPS // 2, step, 0)

        for r in range(3):
            acc_v[pl.ds(r * LANES, LANES)] = (
                acc_v[pl.ds(r * LANES, LANES)]
                + acc_v[pl.ds(PART + r * LANES, LANES)]
            )

        h_res = pltpu.async_copy(res_v, res_hbm.at[pl.ds(base, CHUNK)], sem_out)

        pltpu.sync_copy(acc_v.at[pl.ds(0, PART)], shared.at[pl.ds(wid * PART, PART)])
        plsc.subcore_barrier()

        @pl.when(wid == 0)
        def _():
            pltpu.sync_copy(shared, all_v)
            cnt = zeros_f
            s0 = zeros_f
            s1 = zeros_f
            for t in range(NUM_TILES):
                cnt = cnt + all_v[pl.ds(t * PART, LANES)]
                s0 = s0 + all_v[pl.ds(t * PART + LANES, LANES)]
                s1 = s1 + all_v[pl.ds(t * PART + 2 * LANES, LANES)]
            valid = iota < NUM_CLASSES
            c0 = cen0
            c1 = cen1
            scale = ALPHA / (cnt + 1.0)
            n0 = c0 - (cnt * c0 - s0) * scale
            n1 = c1 - (cnt * c1 - s1) * scale
            plsc.store_scatter(nc_v, [iota], n0, mask=valid)
            plsc.store_scatter(nc_v, [iota + NUM_CLASSES], n1, mask=valid)
            pltpu.sync_copy(nc_v.at[pl.ds(0, CEN)], nc_hbm)

        h_res.wait()

    return k


_sc_center_loss = _make_kernel()


@jax.jit
def kernel(features, labels, centers):
    cen_pad = jnp.pad(centers.T, ((0, 0), (0, LANES - NUM_CLASSES)))
    res, nc = _sc_center_loss(
        features.T.reshape(-1), labels.reshape(-1), cen_pad.reshape(-1)
    )
    return res.reshape(-1, 1), nc.reshape(FEAT_DIM, NUM_CLASSES).T


# speedup vs baseline: 1.0815x; 1.0017x over previous
import functools

import jax
import jax.numpy as jnp
from jax import lax
from jax.experimental import pallas as pl
from jax.experimental.pallas import tpu as pltpu
from jax.experimental.pallas import tpu_sc as plsc

NUM_CLASSES = 10
FEAT_DIM = 2
ALPHA = 0.5
BATCH = 16384

NUM_TILES = 16
CHUNK = BATCH // NUM_TILES
LANES = 16
STEPS = CHUNK // LANES
PART = 3 * LANES
CEN = NUM_CLASSES * FEAT_DIM


def _permute(vals, idx):
    return jnp.take_along_axis(
        vals, idx, axis=0, mode=lax.GatherScatterMode.PROMISE_IN_BOUNDS
    )


def _make_kernel():
    mesh = plsc.VectorSubcoreMesh(
        core_axis_name="c", subcore_axis_name="s", num_cores=1
    )

    @functools.partial(
        pl.kernel,
        mesh=mesh,
        compiler_params=pltpu.CompilerParams(needs_layout_passes=False),
        out_type=[
            jax.ShapeDtypeStruct((BATCH,), jnp.float32),
            jax.ShapeDtypeStruct((CEN,), jnp.float32),
        ],
        scratch_types=[
            pltpu.VMEM((CHUNK,), jnp.int32),
            pltpu.VMEM((CHUNK * FEAT_DIM,), jnp.float32),
            pltpu.VMEM((CHUNK,), jnp.float32),
            pltpu.VMEM((2 * LANES,), jnp.float32),
            pltpu.VMEM((2 * PART,), jnp.float32),
            pltpu.VMEM((NUM_TILES * PART,), jnp.float32),
            pltpu.VMEM((2 * LANES,), jnp.float32),
            pltpu.VMEM_SHARED((NUM_TILES * PART,), jnp.float32),
            pltpu.SemaphoreType.DMA,
            pltpu.SemaphoreType.DMA,
        ],
    )
    def k(feat_hbm, lab_hbm, res_hbm, nc_hbm,
          lab_v, feat_v, res_v, cen_v, acc_v, all_v, nc_v, shared,
          sem_in, sem_out):
        wid = lax.axis_index("s")
        base = wid * CHUNK

        h_lab = pltpu.async_copy(lab_hbm.at[pl.ds(base, CHUNK)], lab_v, sem_in)
        h_f0 = pltpu.async_copy(
            feat_hbm.at[pl.ds(base, CHUNK)], feat_v.at[pl.ds(0, CHUNK)], sem_in
        )
        h_f1 = pltpu.async_copy(
            feat_hbm.at[pl.ds(BATCH + base, CHUNK)],
            feat_v.at[pl.ds(CHUNK, CHUNK)],
            sem_in,
        )
        h_cen = pltpu.async_copy(
            feat_hbm.at[pl.ds(BATCH * FEAT_DIM, 2 * LANES)], cen_v, sem_in
        )
        h_lab.wait()
        h_f0.wait()
        h_f1.wait()
        h_cen.wait()

        iota = lax.iota(jnp.int32, LANES)
        cen0 = cen_v[pl.ds(0, LANES)]
        cen1 = cen_v[pl.ds(LANES, LANES)]
        ones_f = jnp.ones((LANES,), jnp.float32)
        zeros_f = jnp.zeros((LANES,), jnp.float32)

        for r in range(6):
            acc_v[pl.ds(r * LANES, LANES)] = zeros_f

        def step(j, _):
            for u in range(2):
                off = pl.multiple_of(j * (2 * LANES) + u * LANES, LANES)
                bank = u * PART
                lab = lab_v[pl.ds(off, LANES)]
                f0 = feat_v[pl.ds(off, LANES)]
                f1 = feat_v[pl.ds(CHUNK + off, LANES)]
                g0 = _permute(cen0, lab)
                g1 = _permute(cen1, lab)
                d0 = f0 - g0
                d1 = f1 - g1
                res_v[pl.ds(off, LANES)] = d0 * d0 + d1 * d1
                plsc.addupdate_scatter(acc_v, [lab + bank], ones_f)
                plsc.addupdate_scatter(acc_v, [lab + (bank + LANES)], f0)
                plsc.addupdate_scatter(acc_v, [lab + (bank + 2 * LANES)], f1)
            return 0

        lax.fori_loop(0, STEPS // 2, step, 0)

        for r in range(3):
            acc_v[pl.ds(r * LANES, LANES)] = (
                acc_v[pl.ds(r * LANES, LANES)]
                + acc_v[pl.ds(PART + r * LANES, LANES)]
            )

        h_res = pltpu.async_copy(res_v, res_hbm.at[pl.ds(base, CHUNK)], sem_out)

        pltpu.sync_copy(acc_v.at[pl.ds(0, PART)], shared.at[pl.ds(wid * PART, PART)])
        plsc.subcore_barrier()

        @pl.when(wid == 0)
        def _():
            pltpu.sync_copy(shared, all_v)
            cnt = zeros_f
            s0 = zeros_f
            s1 = zeros_f
            for t in range(NUM_TILES):
                cnt = cnt + all_v[pl.ds(t * PART, LANES)]
                s0 = s0 + all_v[pl.ds(t * PART + LANES, LANES)]
                s1 = s1 + all_v[pl.ds(t * PART + 2 * LANES, LANES)]
            valid = iota < NUM_CLASSES
            c0 = cen0
            c1 = cen1
            scale = ALPHA / (cnt + 1.0)
            n0 = c0 - (cnt * c0 - s0) * scale
            n1 = c1 - (cnt * c1 - s1) * scale
            plsc.store_scatter(nc_v, [iota], n0, mask=valid)
            plsc.store_scatter(nc_v, [iota + NUM_CLASSES], n1, mask=valid)
            pltpu.sync_copy(nc_v.at[pl.ds(0, CEN)], nc_hbm)

        h_res.wait()

    return k


_sc_center_loss = _make_kernel()


@jax.jit
def kernel(features, labels, centers):
    cen_pad = jnp.pad(centers.T, ((0, 0), (0, LANES - NUM_CLASSES)))
    packed = jnp.concatenate([features.T.reshape(-1), cen_pad.reshape(-1)])
    res, nc = _sc_center_loss(packed, labels.reshape(-1))
    return res.reshape(-1, 1), nc.reshape(FEAT_DIM, NUM_CLASSES).T
